# baseline (speedup 1.0000x reference)
import jax
import jax.numpy as jnp
from jax.experimental import pallas as pl


def _copy_body(x_ref, o_ref):
    o_ref[...] = x_ref[...]


def _pallas_copy(x, rows_per_block):
    n, d = x.shape
    assert n % rows_per_block == 0
    return pl.pallas_call(
        _copy_body,
        grid=(n // rows_per_block,),
        in_specs=[pl.BlockSpec((rows_per_block, d), lambda i: (i, 0))],
        out_specs=pl.BlockSpec((rows_per_block, d), lambda i: (i, 0)),
        out_shape=jax.ShapeDtypeStruct((n, d), x.dtype),
    )(x)


def kernel(user_emb, item_emb):
    u = _pallas_copy(user_emb, 10000)
    i = _pallas_copy(item_emb, 20000)
    return (u, i)
